# Initial kernel scaffold; baseline (speedup 1.0000x reference)
import jax
import jax.numpy as jnp
from jax.experimental import pallas as pl


def kernel(queries, keys, values, attn_mask):
    raise NotImplementedError("write your pallas kernel here")


# speedup vs baseline: 2.4558x; 2.4558x over previous
import functools
import numpy as np

import jax
import jax.numpy as jnp
from jax import lax
from jax.experimental import pallas as pl
from jax.experimental.pallas import tpu as pltpu

N1 = 64
N2 = 64
L = N1 * N2
TOPK = 8
DC = 128
DCI = 32
T = 512
NB = L // T
_PREC = lax.Precision.HIGHEST


def _tables():
    n = np.arange(64)
    f = np.exp(-2j * np.pi * np.outer(n, n) / 64.0)
    tw = np.exp(-2j * np.pi * np.outer(n, n) / float(L))
    return (
        jnp.asarray(f.real, jnp.float32), jnp.asarray(f.imag, jnp.float32),
        jnp.asarray(tw.real, jnp.float32), jnp.asarray(tw.imag, jnp.float32),
    )


def _dot(a, b):
    return jnp.dot(a, b, preferred_element_type=jnp.float32, precision=_PREC)


def _dot_t(a, b):
    return lax.dot_general(a, b, (((0,), (0,)), ((), ())),
                           preferred_element_type=jnp.float32,
                           precision=_PREC)


def _spectrum_kernel(q_ref, k_ref, fr_ref, fi_ref, twr_ref, twi_ref, s_ref):
    dc = pl.program_id(1)
    fr = fr_ref[...]
    fi = fi_ref[...]
    twr = twr_ref[...][:, None, :]
    twi = twi_ref[...][:, None, :]

    def fwd(x3):
        xm = x3.reshape(N2 * DCI, N1)
        br = _dot(xm, fr).reshape(N2, DCI, N1)
        bi = _dot(xm, fi).reshape(N2, DCI, N1)
        cr = (br * twr - bi * twi).reshape(N2, DCI * N1)
        ci = (br * twi + bi * twr).reshape(N2, DCI * N1)
        xr = _dot(fr, cr) - _dot(fi, ci)
        xi = _dot(fr, ci) + _dot(fi, cr)
        return xr, xi

    sr = jnp.zeros((N1, N2), jnp.float32)
    si = jnp.zeros((N1, N2), jnp.float32)
    for c in range(DC // DCI):
        sl = slice(c * DCI, (c + 1) * DCI)
        qr, qi = fwd(q_ref[0, :, sl, :])
        kr, ki = fwd(k_ref[0, :, sl, :])
        pr = qr * kr + qi * ki
        pi = qi * kr - qr * ki
        sr = sr + pr.reshape(N2, DCI, N1).sum(axis=1)
        si = si + pi.reshape(N2, DCI, N1).sum(axis=1)

    @pl.when(dc == 0)
    def _():
        s_ref[0, 0] = sr
        s_ref[0, 1] = si

    @pl.when(dc != 0)
    def _():
        s_ref[0, 0] += sr
        s_ref[0, 1] += si


def _select_kernel(s_ref, fr_ref, fi_ref, twr_ref, twi_ref, idx_ref, w_ref):
    B = s_ref.shape[0]
    fr = fr_ref[...]
    fi = fi_ref[...]
    twr = twr_ref[...]
    twi = twi_ref[...]
    scale = 1.0 / (float(L) * 768.0)

    mvs = []
    for b in range(B):
        srm = s_ref[b, 0]
        sim = s_ref[b, 1]
        gr = _dot_t(srm, fr) + _dot_t(sim, fi)
        gi = _dot_t(sim, fr) - _dot_t(srm, fi)
        gpr = gr * twr + gi * twi
        gpi = gi * twr - gr * twi
        r = _dot(fr, gpr) + _dot(fi, gpi)
        mvs.append(r * scale)

    g = mvs[0]
    for b in range(1, B):
        g = g + mvs[b]
    g = g * (1.0 / B)

    ii = (lax.broadcasted_iota(jnp.int32, (N1, N2), 0) * N2
          + lax.broadcasted_iota(jnp.int32, (N1, N2), 1))
    lane = lax.broadcasted_iota(jnp.int32, (1, 128), 1)

    cur = g
    wvecs = [jnp.zeros((1, 128), jnp.float32) for _ in range(B)]
    for i in range(TOPK):
        m = jnp.max(cur)
        flat = jnp.min(jnp.where(cur == m, ii, L))
        idx_ref[i:i + 1, :] = flat * jnp.ones((1, 128), jnp.int32)
        sel = ii == flat
        for b in range(B):
            wb = jnp.sum(jnp.where(sel, mvs[b], 0.0))
            wvecs[b] = wvecs[b] + jnp.where(lane == i, wb, 0.0)
        cur = jnp.where(sel, -jnp.inf, cur)

    for b in range(B):
        wm = jnp.where(lane < TOPK, wvecs[b], -jnp.inf)
        mx = jnp.max(wm)
        ev = jnp.exp(wm - mx)
        sv = jnp.sum(ev)
        pv = ev / sv
        for i in range(TOPK):
            val = jnp.sum(jnp.where(lane == i, pv, 0.0))
            w_ref[b, i:i + 1, :] = val * jnp.ones((1, 128), jnp.float32)


def _agg_kernel(d_ref, w_ref, v1_ref, v2_ref, o_ref):
    i = pl.program_id(2)
    b = pl.program_id(0)
    d = d_ref[i]
    sh = d - (d // T) * T
    x = jnp.concatenate([v1_ref[0], v2_ref[0]], axis=0)
    rows = pltpu.roll(x, 2 * T - sh, 0)[:T]
    w = w_ref[b, i]

    @pl.when(i == 0)
    def _():
        o_ref[0] = w * rows

    @pl.when(i != 0)
    def _():
        o_ref[0] += w * rows


@jax.jit
def kernel(queries, keys, values, attn_mask):
    del attn_mask
    B, Lq, H, E = queries.shape
    D = H * E
    q4 = queries.reshape(B, N1, N2, D).transpose(0, 2, 3, 1)
    k4 = keys.reshape(B, N1, N2, D).transpose(0, 2, 3, 1)
    fr, fi, twr, twi = _tables()

    nd = D // DC
    s = pl.pallas_call(
        _spectrum_kernel,
        grid=(B, nd),
        in_specs=[
            pl.BlockSpec((1, N2, DC, N1), lambda b, d: (b, 0, d, 0)),
            pl.BlockSpec((1, N2, DC, N1), lambda b, d: (b, 0, d, 0)),
            pl.BlockSpec((N1, N2), lambda b, d: (0, 0)),
            pl.BlockSpec((N1, N2), lambda b, d: (0, 0)),
            pl.BlockSpec((N1, N2), lambda b, d: (0, 0)),
            pl.BlockSpec((N1, N2), lambda b, d: (0, 0)),
        ],
        out_specs=pl.BlockSpec((1, 2, N1, N2), lambda b, d: (b, 0, 0, 0)),
        out_shape=jax.ShapeDtypeStruct((B, 2, N1, N2), jnp.float32),
        compiler_params=pltpu.CompilerParams(
            dimension_semantics=("parallel", "arbitrary")),
    )(q4, k4, fr, fi, twr, twi)

    idx_pad, w_pad = pl.pallas_call(
        _select_kernel,
        in_specs=[
            pl.BlockSpec((B, 2, N1, N2), lambda: (0, 0, 0, 0)),
            pl.BlockSpec((N1, N2), lambda: (0, 0)),
            pl.BlockSpec((N1, N2), lambda: (0, 0)),
            pl.BlockSpec((N1, N2), lambda: (0, 0)),
            pl.BlockSpec((N1, N2), lambda: (0, 0)),
        ],
        out_specs=[
            pl.BlockSpec((TOPK, 128), lambda: (0, 0)),
            pl.BlockSpec((B, TOPK, 128), lambda: (0, 0, 0)),
        ],
        out_shape=[
            jax.ShapeDtypeStruct((TOPK, 128), jnp.int32),
            jax.ShapeDtypeStruct((B, TOPK, 128), jnp.float32),
        ],
    )(s, fr, fi, twr, twi)

    delays = idx_pad[:, 0]
    weights = w_pad[:, :, 0]
    v3 = values.reshape(B, L, D)

    grid_spec = pltpu.PrefetchScalarGridSpec(
        num_scalar_prefetch=2,
        grid=(B, NB, TOPK),
        in_specs=[
            pl.BlockSpec((1, T, D), lambda b, j, i, dref, wref:
                         (b, (j + dref[i] // T) % NB, 0)),
            pl.BlockSpec((1, T, D), lambda b, j, i, dref, wref:
                         (b, (j + dref[i] // T + 1) % NB, 0)),
        ],
        out_specs=pl.BlockSpec((1, T, D), lambda b, j, i, dref, wref:
                               (b, j, 0)),
    )
    out = pl.pallas_call(
        _agg_kernel,
        grid_spec=grid_spec,
        out_shape=jax.ShapeDtypeStruct((B, L, D), jnp.float32),
        compiler_params=pltpu.CompilerParams(
            dimension_semantics=("parallel", "parallel", "arbitrary")),
    )(delays, weights, v3, v3)

    return out.reshape(B, L, H, E)


# speedup vs baseline: 2.8851x; 1.1748x over previous
import functools
import numpy as np

import jax
import jax.numpy as jnp
from jax import lax
from jax.experimental import pallas as pl
from jax.experimental.pallas import tpu as pltpu

N1 = 64
N2 = 64
L = N1 * N2
TOPK = 8
DC = 128
DCI = 32
T = 512
NB = L // T
_PREC = lax.Precision.HIGHEST


def _tables():
    n = np.arange(64)
    f = np.exp(-2j * np.pi * np.outer(n, n) / 64.0)
    tw = np.exp(-2j * np.pi * np.outer(n, n) / float(L))
    return (
        jnp.asarray(f.real, jnp.float32), jnp.asarray(f.imag, jnp.float32),
        jnp.asarray(tw.real, jnp.float32), jnp.asarray(tw.imag, jnp.float32),
    )


def _dot(a, b):
    return jnp.dot(a, b, preferred_element_type=jnp.float32, precision=_PREC)


def _dot_t(a, b):
    return lax.dot_general(a, b, (((0,), (0,)), ((), ())),
                           preferred_element_type=jnp.float32,
                           precision=_PREC)


def _spectrum_kernel(q_ref, k_ref, fr_ref, fi_ref, twr_ref, twi_ref, s_ref):
    dc = pl.program_id(1)
    fr = fr_ref[...]
    fi = fi_ref[...]
    twr = twr_ref[...][:, None, :]
    twi = twi_ref[...][:, None, :]

    def fwd(x3):
        xm = x3.reshape(N2 * DCI, N1)
        br = _dot(xm, fr).reshape(N2, DCI, N1)
        bi = _dot(xm, fi).reshape(N2, DCI, N1)
        cr = (br * twr - bi * twi).reshape(N2, DCI * N1)
        ci = (br * twi + bi * twr).reshape(N2, DCI * N1)
        xr = _dot(fr, cr) - _dot(fi, ci)
        xi = _dot(fr, ci) + _dot(fi, cr)
        return xr, xi

    sr = jnp.zeros((N1, N2), jnp.float32)
    si = jnp.zeros((N1, N2), jnp.float32)
    for c in range(DC // DCI):
        sl = slice(c * DCI, (c + 1) * DCI)
        qr, qi = fwd(q_ref[0, :, sl, :])
        kr, ki = fwd(k_ref[0, :, sl, :])
        pr = qr * kr + qi * ki
        pi = qi * kr - qr * ki
        sr = sr + pr.reshape(N2, DCI, N1).sum(axis=1)
        si = si + pi.reshape(N2, DCI, N1).sum(axis=1)

    @pl.when(dc == 0)
    def _():
        s_ref[0, 0] = sr
        s_ref[0, 1] = si

    @pl.when(dc != 0)
    def _():
        s_ref[0, 0] += sr
        s_ref[0, 1] += si


def _select_kernel(s_ref, fr_ref, fi_ref, twr_ref, twi_ref, idx_ref, w_ref):
    B = s_ref.shape[0]
    fr = fr_ref[...]
    fi = fi_ref[...]
    twr = twr_ref[...]
    twi = twi_ref[...]
    scale = 1.0 / (float(L) * 768.0)

    mvs = []
    for b in range(B):
        srm = s_ref[b, 0]
        sim = s_ref[b, 1]
        gr = _dot_t(srm, fr) + _dot_t(sim, fi)
        gi = _dot_t(sim, fr) - _dot_t(srm, fi)
        gpr = gr * twr + gi * twi
        gpi = gi * twr - gr * twi
        r = _dot(fr, gpr) + _dot(fi, gpi)
        mvs.append(r * scale)

    g = mvs[0]
    for b in range(1, B):
        g = g + mvs[b]
    g = g * (1.0 / B)

    ii = (lax.broadcasted_iota(jnp.int32, (N1, N2), 0) * N2
          + lax.broadcasted_iota(jnp.int32, (N1, N2), 1))
    lane = lax.broadcasted_iota(jnp.int32, (1, 128), 1)

    cur = g
    wvecs = [jnp.zeros((1, 128), jnp.float32) for _ in range(B)]
    for i in range(TOPK):
        m = jnp.max(cur)
        flat = jnp.min(jnp.where(cur == m, ii, L))
        idx_ref[i:i + 1, :] = flat * jnp.ones((1, 128), jnp.int32)
        sel = ii == flat
        for b in range(B):
            wb = jnp.sum(jnp.where(sel, mvs[b], 0.0))
            wvecs[b] = wvecs[b] + jnp.where(lane == i, wb, 0.0)
        cur = jnp.where(sel, -jnp.inf, cur)

    for b in range(B):
        wm = jnp.where(lane < TOPK, wvecs[b], -jnp.inf)
        mx = jnp.max(wm)
        ev = jnp.exp(wm - mx)
        sv = jnp.sum(ev)
        pv = ev / sv
        for i in range(TOPK):
            val = jnp.sum(jnp.where(lane == i, pv, 0.0))
            w_ref[b, i:i + 1, :] = val * jnp.ones((1, 128), jnp.float32)


def _agg_kernel(d_ref, w_ref, vp_ref, o_ref, buf_ref, sem_ref):
    b = pl.program_id(0)
    j = pl.program_id(1)
    base = j * T

    def _copy(i, slot):
        st = lax.rem(base + d_ref[i], L)
        st_a = (st // 8) * 8
        return pltpu.make_async_copy(
            vp_ref.at[b, pl.ds(st_a, T + 8), :], buf_ref.at[slot],
            sem_ref.at[slot])

    _copy(0, 0).start()
    for i in range(TOPK):
        slot = i % 2
        if i + 1 < TOPK:
            _copy(i + 1, (i + 1) % 2).start()
        _copy(i, slot).wait()
        st = lax.rem(base + d_ref[i], L)
        sh = st - (st // 8) * 8
        rows = pltpu.roll(buf_ref[slot], (T + 8) - sh, 0)[:T]
        w = w_ref[b, i]
        if i == 0:
            o_ref[0] = w * rows
        else:
            o_ref[0] += w * rows


@jax.jit
def kernel(queries, keys, values, attn_mask):
    del attn_mask
    B, Lq, H, E = queries.shape
    D = H * E
    q4 = queries.reshape(B, N1, N2, D).transpose(0, 2, 3, 1)
    k4 = keys.reshape(B, N1, N2, D).transpose(0, 2, 3, 1)
    fr, fi, twr, twi = _tables()

    nd = D // DC
    s = pl.pallas_call(
        _spectrum_kernel,
        grid=(B, nd),
        in_specs=[
            pl.BlockSpec((1, N2, DC, N1), lambda b, d: (b, 0, d, 0)),
            pl.BlockSpec((1, N2, DC, N1), lambda b, d: (b, 0, d, 0)),
            pl.BlockSpec((N1, N2), lambda b, d: (0, 0)),
            pl.BlockSpec((N1, N2), lambda b, d: (0, 0)),
            pl.BlockSpec((N1, N2), lambda b, d: (0, 0)),
            pl.BlockSpec((N1, N2), lambda b, d: (0, 0)),
        ],
        out_specs=pl.BlockSpec((1, 2, N1, N2), lambda b, d: (b, 0, 0, 0)),
        out_shape=jax.ShapeDtypeStruct((B, 2, N1, N2), jnp.float32),
        compiler_params=pltpu.CompilerParams(
            dimension_semantics=("parallel", "arbitrary")),
    )(q4, k4, fr, fi, twr, twi)

    idx_pad, w_pad = pl.pallas_call(
        _select_kernel,
        in_specs=[
            pl.BlockSpec((B, 2, N1, N2), lambda: (0, 0, 0, 0)),
            pl.BlockSpec((N1, N2), lambda: (0, 0)),
            pl.BlockSpec((N1, N2), lambda: (0, 0)),
            pl.BlockSpec((N1, N2), lambda: (0, 0)),
            pl.BlockSpec((N1, N2), lambda: (0, 0)),
        ],
        out_specs=[
            pl.BlockSpec((TOPK, 128), lambda: (0, 0)),
            pl.BlockSpec((B, TOPK, 128), lambda: (0, 0, 0)),
        ],
        out_shape=[
            jax.ShapeDtypeStruct((TOPK, 128), jnp.int32),
            jax.ShapeDtypeStruct((B, TOPK, 128), jnp.float32),
        ],
    )(s, fr, fi, twr, twi)

    delays = idx_pad[:, 0]
    weights = w_pad[:, :, 0]
    v3 = values.reshape(B, L, D)
    vpad = jnp.concatenate([v3, v3[:, :T]], axis=1)

    grid_spec = pltpu.PrefetchScalarGridSpec(
        num_scalar_prefetch=2,
        grid=(B, NB),
        in_specs=[pl.BlockSpec(memory_space=pl.ANY)],
        out_specs=pl.BlockSpec((1, T, D), lambda b, j, dref, wref: (b, j, 0)),
        scratch_shapes=[
            pltpu.VMEM((2, T + 8, D), jnp.float32),
            pltpu.SemaphoreType.DMA((2,)),
        ],
    )
    out = pl.pallas_call(
        _agg_kernel,
        grid_spec=grid_spec,
        out_shape=jax.ShapeDtypeStruct((B, L, D), jnp.float32),
        compiler_params=pltpu.CompilerParams(
            dimension_semantics=("parallel", "parallel")),
    )(delays, weights, vpad)

    return out.reshape(B, L, H, E)


# speedup vs baseline: 3.7623x; 1.3040x over previous
import functools
import numpy as np

import jax
import jax.numpy as jnp
from jax import lax
from jax.experimental import pallas as pl
from jax.experimental.pallas import tpu as pltpu

N1 = 64
N2 = 64
L = N1 * N2
TOPK = 8
DC = 128
DCI = 64
T = 512
NB = L // T


def _tables():
    n = np.arange(64)
    f = np.exp(-2j * np.pi * np.outer(n, n) / 64.0)
    tw = np.exp(-2j * np.pi * np.outer(n, n) / float(L))
    return (
        jnp.asarray(f.real, jnp.float32), jnp.asarray(f.imag, jnp.float32),
        jnp.asarray(tw.real, jnp.float32), jnp.asarray(tw.imag, jnp.float32),
    )


def _split(a):
    hi = a.astype(jnp.bfloat16)
    lo = (a - hi.astype(jnp.float32)).astype(jnp.bfloat16)
    return hi, lo


def _dot3s(asp, bsp, dims):
    ah, al = asp
    bh, bl = bsp
    d = functools.partial(lax.dot_general, dimension_numbers=dims,
                          preferred_element_type=jnp.float32)
    return (d(ah, bl) + d(al, bh)) + d(ah, bh)


def _dot3(a, b, dims):
    return _dot3s(_split(a), _split(b), dims)


def _dot(a, b):
    return _dot3(a, b, (((1,), (0,)), ((), ())))


def _dot_t(a, b):
    return _dot3(a, b, (((0,), (0,)), ((), ())))


def _spectrum_kernel(q_ref, k_ref, fr_ref, fi_ref, twr_ref, twi_ref, s_ref):
    dc = pl.program_id(1)
    fr = fr_ref[...]
    fi = fi_ref[...]
    twr = twr_ref[...][:, None, :]
    twi = twi_ref[...][:, None, :]
    frs = _split(fr)
    fis = _split(fi)
    dims_n = (((1,), (0,)), ((), ()))

    def fwd(x3):
        xs = _split(x3.reshape(N2 * DCI, N1))
        br = _dot3s(xs, frs, dims_n).reshape(N2, DCI, N1)
        bi = _dot3s(xs, fis, dims_n).reshape(N2, DCI, N1)
        crs = _split((br * twr - bi * twi).reshape(N2, DCI * N1))
        cis = _split((br * twi + bi * twr).reshape(N2, DCI * N1))
        xr = _dot3s(frs, crs, dims_n) - _dot3s(fis, cis, dims_n)
        xi = _dot3s(frs, cis, dims_n) + _dot3s(fis, crs, dims_n)
        return xr, xi

    sr = jnp.zeros((N1, N2), jnp.float32)
    si = jnp.zeros((N1, N2), jnp.float32)
    for c in range(DC // DCI):
        sl = slice(c * DCI, (c + 1) * DCI)
        qr, qi = fwd(q_ref[0, :, sl, :])
        kr, ki = fwd(k_ref[0, :, sl, :])
        pr = qr * kr + qi * ki
        pi = qi * kr - qr * ki
        sr = sr + pr.reshape(N2, DCI, N1).sum(axis=1)
        si = si + pi.reshape(N2, DCI, N1).sum(axis=1)

    @pl.when(dc == 0)
    def _():
        s_ref[0, 0] = sr
        s_ref[0, 1] = si

    @pl.when(dc != 0)
    def _():
        s_ref[0, 0] += sr
        s_ref[0, 1] += si


def _select_body(s_ref, fr_ref, fi_ref, twr_ref, twi_ref, idx_ref, w_ref):
    B = s_ref.shape[0]
    fr = fr_ref[...]
    fi = fi_ref[...]
    twr = twr_ref[...]
    twi = twi_ref[...]
    scale = 1.0 / (float(L) * 768.0)

    mvs = []
    for b in range(B):
        srm = s_ref[b, 0]
        sim = s_ref[b, 1]
        gr = _dot_t(srm, fr) + _dot_t(sim, fi)
        gi = _dot_t(sim, fr) - _dot_t(srm, fi)
        gpr = gr * twr + gi * twi
        gpi = gi * twr - gr * twi
        r = _dot(fr, gpr) + _dot(fi, gpi)
        mvs.append(r * scale)

    g = mvs[0]
    for b in range(1, B):
        g = g + mvs[b]
    g = g * (1.0 / B)

    ii = (lax.broadcasted_iota(jnp.int32, (N1, N2), 0) * N2
          + lax.broadcasted_iota(jnp.int32, (N1, N2), 1))
    lane = lax.broadcasted_iota(jnp.int32, (1, 128), 1)

    cur = g
    wvecs = [jnp.zeros((1, 128), jnp.float32) for _ in range(B)]
    for i in range(TOPK):
        m = jnp.max(cur)
        flat = jnp.min(jnp.where(cur == m, ii, L))
        idx_ref[i] = flat
        sel = ii == flat
        for b in range(B):
            wb = jnp.sum(jnp.where(sel, mvs[b], 0.0))
            wvecs[b] = wvecs[b] + jnp.where(lane == i, wb, 0.0)
        cur = jnp.where(sel, -jnp.inf, cur)

    for b in range(B):
        wm = jnp.where(lane < TOPK, wvecs[b], -jnp.inf)
        mx = jnp.max(wm)
        ev = jnp.exp(wm - mx)
        sv = jnp.sum(ev)
        pv = ev / sv
        for i in range(TOPK):
            w_ref[b, i] = jnp.sum(jnp.where(lane == i, pv, 0.0))


def _agg_kernel(s_ref, fr_ref, fi_ref, twr_ref, twi_ref, v_ref, wr_ref,
                o_ref, buf_ref, sem_ref, idx_ref, w_ref):
    b = pl.program_id(0)
    j = pl.program_id(1)
    base = j * T
    lim = L - (T + 8)

    @pl.when(jnp.logical_and(b == 0, j == 0))
    def _():
        _select_body(s_ref, fr_ref, fi_ref, twr_ref, twi_ref, idx_ref, w_ref)

    def _start(i, slot):
        st = lax.rem(base + idx_ref[i], L)
        st_a = (st // 8) * 8

        @pl.when(st_a <= lim)
        def _():
            pltpu.make_async_copy(
                v_ref.at[b, pl.ds(st_a, T + 8), :], buf_ref.at[slot],
                sem_ref.at[slot]).start()

        @pl.when(st_a > lim)
        def _():
            pltpu.make_async_copy(
                wr_ref.at[b, pl.ds(st_a - lim, T + 8), :], buf_ref.at[slot],
                sem_ref.at[slot]).start()

    def _wait(slot):
        pltpu.make_async_copy(
            v_ref.at[b, pl.ds(0, T + 8), :], buf_ref.at[slot],
            sem_ref.at[slot]).wait()

    _start(0, 0)
    for i in range(TOPK):
        slot = i % 2
        if i + 1 < TOPK:
            _start(i + 1, (i + 1) % 2)
        _wait(slot)
        st = lax.rem(base + idx_ref[i], L)
        sh = st - (st // 8) * 8
        rows = pltpu.roll(buf_ref[slot], (T + 8) - sh, 0)[:T]
        w = w_ref[b, i]
        if i == 0:
            o_ref[0] = w * rows
        else:
            o_ref[0] += w * rows


@jax.jit
def kernel(queries, keys, values, attn_mask):
    del attn_mask
    B, Lq, H, E = queries.shape
    D = H * E
    q4 = queries.reshape(B, N1, N2, D).transpose(0, 2, 3, 1)
    k4 = keys.reshape(B, N1, N2, D).transpose(0, 2, 3, 1)
    fr, fi, twr, twi = _tables()

    nd = D // DC
    s = pl.pallas_call(
        _spectrum_kernel,
        grid=(B, nd),
        in_specs=[
            pl.BlockSpec((1, N2, DC, N1), lambda b, d: (b, 0, d, 0)),
            pl.BlockSpec((1, N2, DC, N1), lambda b, d: (b, 0, d, 0)),
            pl.BlockSpec((N1, N2), lambda b, d: (0, 0)),
            pl.BlockSpec((N1, N2), lambda b, d: (0, 0)),
            pl.BlockSpec((N1, N2), lambda b, d: (0, 0)),
            pl.BlockSpec((N1, N2), lambda b, d: (0, 0)),
        ],
        out_specs=pl.BlockSpec((1, 2, N1, N2), lambda b, d: (b, 0, 0, 0)),
        out_shape=jax.ShapeDtypeStruct((B, 2, N1, N2), jnp.float32),
        compiler_params=pltpu.CompilerParams(
            dimension_semantics=("parallel", "arbitrary")),
    )(q4, k4, fr, fi, twr, twi)

    v3 = values.reshape(B, L, D)
    wring = jnp.concatenate([v3[:, L - T - 8:], v3[:, :T + 8]], axis=1)

    out = pl.pallas_call(
        _agg_kernel,
        grid=(B, NB),
        in_specs=[
            pl.BlockSpec((B, 2, N1, N2), lambda b, j: (0, 0, 0, 0)),
            pl.BlockSpec((N1, N2), lambda b, j: (0, 0)),
            pl.BlockSpec((N1, N2), lambda b, j: (0, 0)),
            pl.BlockSpec((N1, N2), lambda b, j: (0, 0)),
            pl.BlockSpec((N1, N2), lambda b, j: (0, 0)),
            pl.BlockSpec(memory_space=pl.ANY),
            pl.BlockSpec(memory_space=pl.ANY),
        ],
        out_specs=pl.BlockSpec((1, T, D), lambda b, j: (b, j, 0)),
        out_shape=jax.ShapeDtypeStruct((B, L, D), jnp.float32),
        scratch_shapes=[
            pltpu.VMEM((2, T + 8, D), jnp.float32),
            pltpu.SemaphoreType.DMA((2,)),
            pltpu.SMEM((TOPK,), jnp.int32),
            pltpu.SMEM((B, TOPK), jnp.float32),
        ],
        compiler_params=pltpu.CompilerParams(
            dimension_semantics=("arbitrary", "arbitrary")),
    )(s, fr, fi, twr, twi, v3, wring)

    return out.reshape(B, L, H, E)
